# Initial kernel scaffold; baseline (speedup 1.0000x reference)
import jax
import jax.numpy as jnp
from jax.experimental import pallas as pl


def kernel(X, W1, b1, Hw, W2, b2):
    raise NotImplementedError("write your pallas kernel here")


# speedup vs baseline: 1.1419x; 1.1419x over previous
import functools

import jax
import jax.numpy as jnp
import numpy as np
from jax.experimental import pallas as pl

INPUT_SIZE = 1024
HIDDEN_SIZE = 4096
NUM_CLASSES = 16384
K = 10
L = 8
NUM_SAMPLED = 1024
BATCH = 2048

_G_NP = np.zeros((L * K, L), dtype=np.float32)
for _l in range(L):
    for _k in range(K):
        _G_NP[_l * K + _k, _l] = float(2 ** _k)


def _sel_kernel(xw_ref, hwt_ref, g_ref, mask_ref):
    proj = jnp.dot(xw_ref[...], hwt_ref[...],
                   preferred_element_type=jnp.float32)
    bits = (proj > 0).astype(jnp.float32)
    codes_f = jnp.dot(bits, g_ref[...],
                      preferred_element_type=jnp.float32,
                      precision=jax.lax.Precision.HIGHEST)
    codes = codes_f.astype(jnp.int32)
    cw_t = codes[BATCH:, :].T
    counts = jnp.zeros((1, HIDDEN_SIZE), dtype=jnp.int32)
    for l in range(L):
        q_col = codes[:BATCH, l:l + 1]
        w_row = cw_t[l:l + 1, :]
        match = (q_col == w_row).astype(jnp.int32)
        counts = counts + jnp.sum(match, axis=0, keepdims=True)
    hidx = jax.lax.broadcasted_iota(jnp.int32, (1, HIDDEN_SIZE), 1)
    keys = counts * HIDDEN_SIZE + (HIDDEN_SIZE - 1 - hidx)
    def body(_, lohi):
        lo, hi = lohi
        mid = (lo + hi) >> 1
        cnt = jnp.sum((keys >= mid).astype(jnp.int32))
        ok = cnt >= NUM_SAMPLED
        return (jnp.where(ok, mid, lo), jnp.where(ok, hi, mid))
    lo, _ = jax.lax.fori_loop(0, 27, body, (jnp.int32(0), jnp.int32(1 << 27)))
    mask_ref[...] = (keys >= lo).astype(jnp.float32)


def _hidden_kernel(x_ref, w1_ref, b1_ref, m_ref, out_ref):
    x = x_ref[...].astype(jnp.bfloat16)
    w = w1_ref[...].astype(jnp.bfloat16)
    acc = jax.lax.dot_general(x, w, (((1,), (1,)), ((), ())),
                              preferred_element_type=jnp.float32)
    acc = acc + b1_ref[...]
    acc = jnp.maximum(acc, 0.0) * m_ref[...]
    out_ref[...] = acc.astype(jnp.bfloat16)


def _out_kernel(h_ref, w2_ref, b2_ref, out_ref):
    w = w2_ref[...].astype(jnp.bfloat16)
    acc = jax.lax.dot_general(h_ref[...], w, (((1,), (1,)), ((), ())),
                              preferred_element_type=jnp.float32)
    out_ref[...] = acc + b2_ref[...]


@jax.jit
def kernel(X, W1, b1, Hw, W2, b2):
    hw_t = Hw.reshape(L * K, INPUT_SIZE).T
    xw = jnp.concatenate([X, W1], axis=0)
    g = jnp.asarray(_G_NP)

    mask = pl.pallas_call(
        _sel_kernel,
        out_shape=jax.ShapeDtypeStruct((1, HIDDEN_SIZE), jnp.float32),
    )(xw, hw_t, g)

    b1r = b1.reshape(1, HIDDEN_SIZE)
    HB = 512
    hlog = pl.pallas_call(
        _hidden_kernel,
        grid=(HIDDEN_SIZE // HB,),
        in_specs=[
            pl.BlockSpec((BATCH, INPUT_SIZE), lambda i: (0, 0)),
            pl.BlockSpec((HB, INPUT_SIZE), lambda i: (i, 0)),
            pl.BlockSpec((1, HB), lambda i: (0, i)),
            pl.BlockSpec((1, HB), lambda i: (0, i)),
        ],
        out_specs=pl.BlockSpec((BATCH, HB), lambda i: (0, i)),
        out_shape=jax.ShapeDtypeStruct((BATCH, HIDDEN_SIZE), jnp.bfloat16),
    )(X, W1, b1r, mask)

    b2r = b2.reshape(1, NUM_CLASSES)
    CB = 512
    out = pl.pallas_call(
        _out_kernel,
        grid=(NUM_CLASSES // CB,),
        in_specs=[
            pl.BlockSpec((BATCH, HIDDEN_SIZE), lambda i: (0, 0)),
            pl.BlockSpec((CB, HIDDEN_SIZE), lambda i: (i, 0)),
            pl.BlockSpec((1, CB), lambda i: (0, i)),
        ],
        out_specs=pl.BlockSpec((BATCH, CB), lambda i: (0, i)),
        out_shape=jax.ShapeDtypeStruct((BATCH, NUM_CLASSES), jnp.float32),
    )(hlog, W2, b2r)
    return out


# speedup vs baseline: 1.2063x; 1.0563x over previous
import functools

import jax
import jax.numpy as jnp
import numpy as np
from jax.experimental import pallas as pl

INPUT_SIZE = 1024
HIDDEN_SIZE = 4096
NUM_CLASSES = 16384
K = 10
L = 8
NUM_SAMPLED = 1024
BATCH = 2048

_G_NP = np.zeros((L * K, L), dtype=np.float32)
for _l in range(L):
    for _k in range(K):
        _G_NP[_l * K + _k, _l] = float(2 ** _k)


def _sel_kernel(xw_ref, hwt_ref, g_ref, mask_ref):
    proj = jnp.dot(xw_ref[...], hwt_ref[...],
                   preferred_element_type=jnp.float32)
    bits = (proj > 0).astype(jnp.float32)
    codes_f = jnp.dot(bits, g_ref[...],
                      preferred_element_type=jnp.float32,
                      precision=jax.lax.Precision.HIGHEST)
    codes = codes_f.astype(jnp.int32)
    cw_t = codes[BATCH:, :].T
    NB = 1 << K
    hist_rows = []
    for l in range(L):
        q_col = codes[:BATCH, l:l + 1]
        oh_q = (q_col == jax.lax.broadcasted_iota(jnp.int32, (BATCH, NB), 1))
        hist_rows.append(jnp.sum(oh_q.astype(jnp.int32), axis=0, keepdims=True))
    hist = jnp.concatenate(hist_rows, axis=0)
    counts_l = jnp.zeros((L, HIDDEN_SIZE), dtype=jnp.int32)
    for c in range(NB // 128):
        src = hist[:, c * 128:(c + 1) * 128]
        idx = jnp.clip(cw_t - c * 128, 0, 127)
        g = jnp.take_along_axis(src, idx, axis=1)
        valid = (cw_t >= c * 128) & (cw_t < (c + 1) * 128)
        counts_l = counts_l + jnp.where(valid, g, 0)
    counts = jnp.sum(counts_l, axis=0, keepdims=True)
    hidx = jax.lax.broadcasted_iota(jnp.int32, (1, HIDDEN_SIZE), 1)
    keys = counts * HIDDEN_SIZE + (HIDDEN_SIZE - 1 - hidx)
    def body(_, lohi):
        lo, hi = lohi
        mid = (lo + hi) >> 1
        cnt = jnp.sum((keys >= mid).astype(jnp.int32))
        ok = cnt >= NUM_SAMPLED
        return (jnp.where(ok, mid, lo), jnp.where(ok, hi, mid))
    lo, _ = jax.lax.fori_loop(0, 27, body, (jnp.int32(0), jnp.int32(1 << 27)))
    mask_ref[...] = (keys >= lo).astype(jnp.float32)


def _hidden_kernel(x_ref, w1_ref, b1_ref, m_ref, out_ref):
    x = x_ref[...].astype(jnp.bfloat16)
    w = w1_ref[...].astype(jnp.bfloat16)
    acc = jax.lax.dot_general(x, w, (((1,), (1,)), ((), ())),
                              preferred_element_type=jnp.float32)
    acc = acc + b1_ref[...]
    acc = jnp.maximum(acc, 0.0) * m_ref[...]
    out_ref[...] = acc.astype(jnp.bfloat16)


def _out_kernel(h_ref, w2_ref, b2_ref, out_ref):
    w = w2_ref[...].astype(jnp.bfloat16)
    acc = jax.lax.dot_general(h_ref[...], w, (((1,), (1,)), ((), ())),
                              preferred_element_type=jnp.float32)
    out_ref[...] = acc + b2_ref[...]


@jax.jit
def kernel(X, W1, b1, Hw, W2, b2):
    hw_t = Hw.reshape(L * K, INPUT_SIZE).T
    xw = jnp.concatenate([X, W1], axis=0)
    g = jnp.asarray(_G_NP)

    mask = pl.pallas_call(
        _sel_kernel,
        out_shape=jax.ShapeDtypeStruct((1, HIDDEN_SIZE), jnp.float32),
    )(xw, hw_t, g)

    b1r = b1.reshape(1, HIDDEN_SIZE)
    HB = 512
    hlog = pl.pallas_call(
        _hidden_kernel,
        grid=(HIDDEN_SIZE // HB,),
        in_specs=[
            pl.BlockSpec((BATCH, INPUT_SIZE), lambda i: (0, 0)),
            pl.BlockSpec((HB, INPUT_SIZE), lambda i: (i, 0)),
            pl.BlockSpec((1, HB), lambda i: (0, i)),
            pl.BlockSpec((1, HB), lambda i: (0, i)),
        ],
        out_specs=pl.BlockSpec((BATCH, HB), lambda i: (0, i)),
        out_shape=jax.ShapeDtypeStruct((BATCH, HIDDEN_SIZE), jnp.bfloat16),
    )(X, W1, b1r, mask)

    b2r = b2.reshape(1, NUM_CLASSES)
    CB = 512
    out = pl.pallas_call(
        _out_kernel,
        grid=(NUM_CLASSES // CB,),
        in_specs=[
            pl.BlockSpec((BATCH, HIDDEN_SIZE), lambda i: (0, 0)),
            pl.BlockSpec((CB, HIDDEN_SIZE), lambda i: (i, 0)),
            pl.BlockSpec((1, CB), lambda i: (0, i)),
        ],
        out_specs=pl.BlockSpec((BATCH, CB), lambda i: (0, i)),
        out_shape=jax.ShapeDtypeStruct((BATCH, NUM_CLASSES), jnp.float32),
    )(hlog, W2, b2r)
    return out


# speedup vs baseline: 1.2688x; 1.0518x over previous
import jax
import jax.numpy as jnp
import numpy as np
from jax.experimental import pallas as pl

INPUT_SIZE = 1024
HIDDEN_SIZE = 4096
NUM_CLASSES = 16384
K = 10
L = 8
NUM_SAMPLED = 1024
BATCH = 2048
DAUG = INPUT_SIZE + 8

_G_NP = np.zeros((L * K, L), dtype=np.float32)
for _l in range(L):
    for _k in range(K):
        _G_NP[_l * K + _k, _l] = float(2 ** _k)


def _sel_kernel(xw_ref, hwt_ref, g_ref, sel_ref):
    proj = jnp.dot(xw_ref[...], hwt_ref[...],
                   preferred_element_type=jnp.float32)
    bits = (proj > 0).astype(jnp.float32)
    codes_f = jnp.dot(bits, g_ref[...],
                      preferred_element_type=jnp.float32,
                      precision=jax.lax.Precision.HIGHEST)
    codes = codes_f.astype(jnp.int32)
    cw_t = codes[BATCH:, :].T
    NB = 1 << K
    hist_rows = []
    for l in range(L):
        q_col = codes[:BATCH, l:l + 1]
        oh_q = (q_col == jax.lax.broadcasted_iota(jnp.int32, (BATCH, NB), 1))
        hist_rows.append(jnp.sum(oh_q.astype(jnp.int32), axis=0, keepdims=True))
    hist = jnp.concatenate(hist_rows, axis=0)
    counts_l = jnp.zeros((L, HIDDEN_SIZE), dtype=jnp.int32)
    for c in range(NB // 128):
        src = hist[:, c * 128:(c + 1) * 128]
        idx = jnp.clip(cw_t - c * 128, 0, 127)
        g = jnp.take_along_axis(src, idx, axis=1)
        valid = (cw_t >= c * 128) & (cw_t < (c + 1) * 128)
        counts_l = counts_l + jnp.where(valid, g, 0)
    counts = jnp.sum(counts_l, axis=0, keepdims=True)
    hidx = jax.lax.broadcasted_iota(jnp.int32, (1, HIDDEN_SIZE), 1)
    keys = counts * HIDDEN_SIZE + (HIDDEN_SIZE - 1 - hidx)
    def body(_, lohi):
        lo, hi = lohi
        mid = (lo + hi) >> 1
        cnt = jnp.sum((keys >= mid).astype(jnp.int32))
        ok = cnt >= NUM_SAMPLED
        return (jnp.where(ok, mid, lo), jnp.where(ok, hi, mid))
    lo, _ = jax.lax.fori_loop(0, 27, body, (jnp.int32(0), jnp.int32(1 << 27)))
    counts_col = jnp.sum(counts_l.T, axis=1, keepdims=True)
    hcol = jax.lax.broadcasted_iota(jnp.int32, (HIDDEN_SIZE, 1), 0)
    keys_col = counts_col * HIDDEN_SIZE + (HIDDEN_SIZE - 1 - hcol)
    mask_col = keys_col >= lo
    mask_bf = mask_col.astype(jnp.bfloat16)
    rank = jnp.zeros((HIDDEN_SIZE, 1), dtype=jnp.float32)
    CH = 1024
    for j in range(HIDDEN_SIZE // CH):
        src_idx = jax.lax.broadcasted_iota(jnp.int32, (HIDDEN_SIZE, CH), 1)
        tri = ((src_idx + j * CH) < hcol).astype(jnp.bfloat16)
        mchunk = mask_bf[j * CH:(j + 1) * CH, :]
        rank = rank + jnp.dot(tri, mchunk,
                              preferred_element_type=jnp.float32)
    rank_i = rank.astype(jnp.int32)
    sidx = jax.lax.broadcasted_iota(jnp.int32, (HIDDEN_SIZE, NUM_SAMPLED), 1)
    sel = (rank_i == sidx) & mask_col
    sel_ref[...] = sel.astype(jnp.bfloat16)


def _hidden_kernel(xa_ref, w1a_ref, sel_ref, out_ref):
    w1a = w1a_ref[...].astype(jnp.bfloat16)
    sel = sel_ref[...]
    w1s_t = jax.lax.dot_general(w1a, sel, (((0,), (0,)), ((), ())),
                                preferred_element_type=jnp.float32)
    w1s_t = w1s_t.astype(jnp.bfloat16)
    xa = xa_ref[...].astype(jnp.bfloat16)
    acc = jax.lax.dot_general(xa, w1s_t, (((1,), (0,)), ((), ())),
                              preferred_element_type=jnp.float32)
    out_ref[...] = jnp.maximum(acc, 0.0).astype(jnp.bfloat16)


def _out_kernel(h_ref, w2_ref, sel_ref, b2_ref, out_ref):
    w2 = w2_ref[...].astype(jnp.bfloat16)
    w2s = jax.lax.dot_general(w2, sel_ref[...], (((1,), (0,)), ((), ())),
                              preferred_element_type=jnp.float32)
    w2s = w2s.astype(jnp.bfloat16)
    acc = jax.lax.dot_general(h_ref[...], w2s, (((1,), (1,)), ((), ())),
                              preferred_element_type=jnp.float32)
    out_ref[...] = acc + b2_ref[...]


@jax.jit
def kernel(X, W1, b1, Hw, W2, b2):
    hw_t = Hw.reshape(L * K, INPUT_SIZE).T
    xw = jnp.concatenate([X, W1], axis=0)
    g = jnp.asarray(_G_NP)

    sel = pl.pallas_call(
        _sel_kernel,
        out_shape=jax.ShapeDtypeStruct((HIDDEN_SIZE, NUM_SAMPLED), jnp.bfloat16),
    )(xw, hw_t, g)

    pad_x = jnp.ones((BATCH, 1), jnp.float32)
    xa = jnp.concatenate(
        [X, pad_x, jnp.zeros((BATCH, DAUG - INPUT_SIZE - 1), jnp.float32)],
        axis=1)
    w1a = jnp.concatenate(
        [W1, b1.reshape(HIDDEN_SIZE, 1),
         jnp.zeros((HIDDEN_SIZE, DAUG - INPUT_SIZE - 1), jnp.float32)],
        axis=1)

    hlog_s = pl.pallas_call(
        _hidden_kernel,
        out_shape=jax.ShapeDtypeStruct((BATCH, NUM_SAMPLED), jnp.bfloat16),
    )(xa, w1a, sel)

    b2r = b2.reshape(1, NUM_CLASSES)
    CB = 512
    out = pl.pallas_call(
        _out_kernel,
        grid=(NUM_CLASSES // CB,),
        in_specs=[
            pl.BlockSpec((BATCH, NUM_SAMPLED), lambda i: (0, 0)),
            pl.BlockSpec((CB, HIDDEN_SIZE), lambda i: (i, 0)),
            pl.BlockSpec((HIDDEN_SIZE, NUM_SAMPLED), lambda i: (0, 0)),
            pl.BlockSpec((1, CB), lambda i: (0, i)),
        ],
        out_specs=pl.BlockSpec((BATCH, CB), lambda i: (0, i)),
        out_shape=jax.ShapeDtypeStruct((BATCH, NUM_CLASSES), jnp.float32),
    )(hlog_s, W2, sel, b2r)
    return out


# speedup vs baseline: 1.2961x; 1.0215x over previous
import jax
import jax.numpy as jnp
import numpy as np
from jax.experimental import pallas as pl

INPUT_SIZE = 1024
HIDDEN_SIZE = 4096
NUM_CLASSES = 16384
K = 10
L = 8
NUM_SAMPLED = 1024
BATCH = 2048
DAUG = INPUT_SIZE + 8

_G_NP = np.zeros((L * K, L), dtype=np.float32)
for _l in range(L):
    for _k in range(K):
        _G_NP[_l * K + _k, _l] = float(2 ** _k)


def _sel_kernel(xw_ref, hwt_ref, g_ref, sel_ref):
    proj = jnp.dot(xw_ref[...], hwt_ref[...],
                   preferred_element_type=jnp.float32)
    bits = (proj > 0).astype(jnp.float32)
    codes_f = jnp.dot(bits, g_ref[...],
                      preferred_element_type=jnp.float32,
                      precision=jax.lax.Precision.HIGHEST)
    codes = codes_f.astype(jnp.int32)
    cw_t = codes[BATCH:, :].T
    NB = 1 << K
    ones_row = jnp.ones((1, BATCH), dtype=jnp.bfloat16)
    hist_rows = []
    for l in range(L):
        q_col = codes[:BATCH, l:l + 1]
        oh_q = (q_col == jax.lax.broadcasted_iota(jnp.int32, (BATCH, NB), 1))
        hrow = jnp.dot(ones_row, oh_q.astype(jnp.bfloat16),
                       preferred_element_type=jnp.float32)
        hist_rows.append(hrow.astype(jnp.int32))
    hist = jnp.concatenate(hist_rows, axis=0)
    counts_l = jnp.zeros((L, HIDDEN_SIZE), dtype=jnp.int32)
    for c in range(NB // 128):
        src = hist[:, c * 128:(c + 1) * 128]
        idx = jnp.clip(cw_t - c * 128, 0, 127)
        g = jnp.take_along_axis(src, idx, axis=1)
        valid = (cw_t >= c * 128) & (cw_t < (c + 1) * 128)
        counts_l = counts_l + jnp.where(valid, g, 0)
    counts = jnp.sum(counts_l, axis=0, keepdims=True)
    hidx = jax.lax.broadcasted_iota(jnp.int32, (1, HIDDEN_SIZE), 1)
    keys = counts * HIDDEN_SIZE + (HIDDEN_SIZE - 1 - hidx)
    def body(_, lohi):
        lo, hi = lohi
        mid = (lo + hi) >> 1
        cnt = jnp.sum((keys >= mid).astype(jnp.int32))
        ok = cnt >= NUM_SAMPLED
        return (jnp.where(ok, mid, lo), jnp.where(ok, hi, mid))
    lo, _ = jax.lax.fori_loop(0, 27, body, (jnp.int32(0), jnp.int32(1 << 27)))
    mask_row = keys >= lo
    mask_bf = mask_row.astype(jnp.bfloat16)
    CH = 1024
    rank_chunks = []
    for j in range(HIDDEN_SIZE // CH):
        hp = jax.lax.broadcasted_iota(jnp.int32, (HIDDEN_SIZE, CH), 0)
        dst = jax.lax.broadcasted_iota(jnp.int32, (HIDDEN_SIZE, CH), 1)
        tri = (hp < (dst + j * CH)).astype(jnp.bfloat16)
        rank_chunks.append(jnp.dot(mask_bf, tri,
                                   preferred_element_type=jnp.float32))
    rank_i = jnp.concatenate(rank_chunks, axis=1).astype(jnp.int32)
    sidx = jax.lax.broadcasted_iota(jnp.int32, (NUM_SAMPLED, HIDDEN_SIZE), 0)
    sel_t = (rank_i == sidx) & mask_row
    sel_ref[...] = sel_t.astype(jnp.bfloat16)


def _hidden_kernel(xa_ref, w1a_ref, sel_ref, out_ref):
    w1a = w1a_ref[...].astype(jnp.bfloat16)
    sel = sel_ref[...]
    w1s_t = jax.lax.dot_general(w1a, sel, (((0,), (1,)), ((), ())),
                                preferred_element_type=jnp.float32)
    w1s_t = w1s_t.astype(jnp.bfloat16)
    xa = xa_ref[...].astype(jnp.bfloat16)
    acc = jax.lax.dot_general(xa, w1s_t, (((1,), (0,)), ((), ())),
                              preferred_element_type=jnp.float32)
    out_ref[...] = jnp.maximum(acc, 0.0).astype(jnp.bfloat16)


def _out_kernel(h_ref, w2_ref, sel_ref, b2_ref, out_ref):
    w2 = w2_ref[...].astype(jnp.bfloat16)
    w2s = jax.lax.dot_general(w2, sel_ref[...], (((1,), (1,)), ((), ())),
                              preferred_element_type=jnp.float32)
    w2s = w2s.astype(jnp.bfloat16)
    acc = jax.lax.dot_general(h_ref[...], w2s, (((1,), (1,)), ((), ())),
                              preferred_element_type=jnp.float32)
    out_ref[...] = acc + b2_ref[...]


@jax.jit
def kernel(X, W1, b1, Hw, W2, b2):
    hw_t = Hw.reshape(L * K, INPUT_SIZE).T
    xw = jnp.concatenate([X, W1], axis=0)
    g = jnp.asarray(_G_NP)

    sel = pl.pallas_call(
        _sel_kernel,
        out_shape=jax.ShapeDtypeStruct((NUM_SAMPLED, HIDDEN_SIZE), jnp.bfloat16),
    )(xw, hw_t, g)

    pad_x = jnp.ones((BATCH, 1), jnp.float32)
    xa = jnp.concatenate(
        [X, pad_x, jnp.zeros((BATCH, DAUG - INPUT_SIZE - 1), jnp.float32)],
        axis=1)
    w1a = jnp.concatenate(
        [W1, b1.reshape(HIDDEN_SIZE, 1),
         jnp.zeros((HIDDEN_SIZE, DAUG - INPUT_SIZE - 1), jnp.float32)],
        axis=1)

    hlog_s = pl.pallas_call(
        _hidden_kernel,
        out_shape=jax.ShapeDtypeStruct((BATCH, NUM_SAMPLED), jnp.bfloat16),
    )(xa, w1a, sel)

    b2r = b2.reshape(1, NUM_CLASSES)
    CB = 512
    out = pl.pallas_call(
        _out_kernel,
        grid=(NUM_CLASSES // CB,),
        in_specs=[
            pl.BlockSpec((BATCH, NUM_SAMPLED), lambda i: (0, 0)),
            pl.BlockSpec((CB, HIDDEN_SIZE), lambda i: (i, 0)),
            pl.BlockSpec((NUM_SAMPLED, HIDDEN_SIZE), lambda i: (0, 0)),
            pl.BlockSpec((1, CB), lambda i: (0, i)),
        ],
        out_specs=pl.BlockSpec((BATCH, CB), lambda i: (0, i)),
        out_shape=jax.ShapeDtypeStruct((BATCH, NUM_CLASSES), jnp.float32),
    )(hlog_s, W2, sel, b2r)
    return out


# speedup vs baseline: 1.5731x; 1.2137x over previous
import jax
import jax.numpy as jnp
import numpy as np
from jax.experimental import pallas as pl

INPUT_SIZE = 1024
HIDDEN_SIZE = 4096
NUM_CLASSES = 16384
K = 10
L = 8
NUM_SAMPLED = 1024
BATCH = 2048
DAUG = INPUT_SIZE + 8

_G_NP = np.zeros((L * K, L), dtype=np.float32)
for _l in range(L):
    for _k in range(K):
        _G_NP[_l * K + _k, _l] = float(2 ** _k)


def _sel_kernel(x_ref, w1_ref, hwt_ref, g_ref, sel_ref):
    proj_q = jnp.dot(x_ref[...], hwt_ref[...],
                     preferred_element_type=jnp.float32)
    proj_w = jnp.dot(w1_ref[...], hwt_ref[...],
                     preferred_element_type=jnp.float32)
    g = g_ref[...]
    codes_q = jnp.dot((proj_q > 0).astype(jnp.float32), g,
                      preferred_element_type=jnp.float32,
                      precision=jax.lax.Precision.HIGHEST).astype(jnp.int32)
    codes_w = jnp.dot((proj_w > 0).astype(jnp.float32), g,
                      preferred_element_type=jnp.float32,
                      precision=jax.lax.Precision.HIGHEST).astype(jnp.int32)
    cw_t = codes_w.T
    NB = 1 << K
    ones_row = jnp.ones((1, BATCH), dtype=jnp.bfloat16)
    hist_rows = []
    for l in range(L):
        q_col = codes_q[:, l:l + 1]
        oh_q = (q_col == jax.lax.broadcasted_iota(jnp.int32, (BATCH, NB), 1))
        hrow = jnp.dot(ones_row, oh_q.astype(jnp.bfloat16),
                       preferred_element_type=jnp.float32)
        hist_rows.append(hrow.astype(jnp.int32))
    hist = jnp.concatenate(hist_rows, axis=0)
    counts_l = jnp.zeros((L, HIDDEN_SIZE), dtype=jnp.int32)
    for c in range(NB // 128):
        src = hist[:, c * 128:(c + 1) * 128]
        idx = jnp.clip(cw_t - c * 128, 0, 127)
        g = jnp.take_along_axis(src, idx, axis=1)
        valid = (cw_t >= c * 128) & (cw_t < (c + 1) * 128)
        counts_l = counts_l + jnp.where(valid, g, 0)
    counts = jnp.sum(counts_l, axis=0, keepdims=True)
    hidx = jax.lax.broadcasted_iota(jnp.int32, (1, HIDDEN_SIZE), 1)
    keys = counts * HIDDEN_SIZE + (HIDDEN_SIZE - 1 - hidx)
    def body(_, lohi):
        lo, hi = lohi
        mid = (lo + hi) >> 1
        cnt = jnp.sum((keys >= mid).astype(jnp.int32))
        ok = cnt >= NUM_SAMPLED
        return (jnp.where(ok, mid, lo), jnp.where(ok, hi, mid))
    lo, _ = jax.lax.fori_loop(0, 27, body, (jnp.int32(0), jnp.int32(1 << 27)))
    mask_row = keys >= lo
    mask_bf = mask_row.astype(jnp.bfloat16)
    CH = 1024
    rank_chunks = []
    for j in range(HIDDEN_SIZE // CH):
        hp = jax.lax.broadcasted_iota(jnp.int32, (HIDDEN_SIZE, CH), 0)
        dst = jax.lax.broadcasted_iota(jnp.int32, (HIDDEN_SIZE, CH), 1)
        tri = (hp < (dst + j * CH)).astype(jnp.bfloat16)
        rank_chunks.append(jnp.dot(mask_bf, tri,
                                   preferred_element_type=jnp.float32))
    rank_i = jnp.concatenate(rank_chunks, axis=1).astype(jnp.int32)
    sidx = jax.lax.broadcasted_iota(jnp.int32, (NUM_SAMPLED, HIDDEN_SIZE), 0)
    sel_t = (rank_i == sidx) & mask_row
    sel_ref[...] = sel_t.astype(jnp.bfloat16)


def _hidden_kernel(x_ref, w1_ref, b1_ref, sel_ref, out_ref):
    w1 = w1_ref[...].astype(jnp.bfloat16)
    sel = sel_ref[...]
    w1s = jax.lax.dot_general(sel, w1, (((1,), (0,)), ((), ())),
                              preferred_element_type=jnp.float32)
    w1s = w1s.astype(jnp.bfloat16)
    b1s = jax.lax.dot_general(b1_ref[...].astype(jnp.bfloat16), sel,
                              (((1,), (1,)), ((), ())),
                              preferred_element_type=jnp.float32)
    x = x_ref[...].astype(jnp.bfloat16)
    acc = jax.lax.dot_general(x, w1s, (((1,), (1,)), ((), ())),
                              preferred_element_type=jnp.float32)
    out_ref[...] = jnp.maximum(acc + b1s, 0.0).astype(jnp.bfloat16)


def _out_kernel(h_ref, w2_ref, sel_ref, b2_ref, out_ref):
    w2 = w2_ref[...].astype(jnp.bfloat16)
    w2s = jax.lax.dot_general(w2, sel_ref[...], (((1,), (1,)), ((), ())),
                              preferred_element_type=jnp.float32)
    w2s = w2s.astype(jnp.bfloat16)
    acc = jax.lax.dot_general(h_ref[...], w2s, (((1,), (1,)), ((), ())),
                              preferred_element_type=jnp.float32)
    out_ref[...] = acc + b2_ref[...]


@jax.jit
def kernel(X, W1, b1, Hw, W2, b2):
    hw_t = Hw.reshape(L * K, INPUT_SIZE).T
    g = jnp.asarray(_G_NP)

    sel = pl.pallas_call(
        _sel_kernel,
        out_shape=jax.ShapeDtypeStruct((NUM_SAMPLED, HIDDEN_SIZE), jnp.bfloat16),
    )(X, W1, hw_t, g)

    b1r = b1.reshape(1, HIDDEN_SIZE)
    hlog_s = pl.pallas_call(
        _hidden_kernel,
        out_shape=jax.ShapeDtypeStruct((BATCH, NUM_SAMPLED), jnp.bfloat16),
    )(X, W1, b1r, sel)

    b2r = b2.reshape(1, NUM_CLASSES)
    CB = 512
    out = pl.pallas_call(
        _out_kernel,
        grid=(NUM_CLASSES // CB,),
        in_specs=[
            pl.BlockSpec((BATCH, NUM_SAMPLED), lambda i: (0, 0)),
            pl.BlockSpec((CB, HIDDEN_SIZE), lambda i: (i, 0)),
            pl.BlockSpec((NUM_SAMPLED, HIDDEN_SIZE), lambda i: (0, 0)),
            pl.BlockSpec((1, CB), lambda i: (0, i)),
        ],
        out_specs=pl.BlockSpec((BATCH, CB), lambda i: (0, i)),
        out_shape=jax.ShapeDtypeStruct((BATCH, NUM_CLASSES), jnp.float32),
    )(hlog_s, W2, sel, b2r)
    return out
